# baseline (device time: 61229 ns/iter reference)
import jax
import jax.numpy as jnp
from jax import lax
from jax.experimental import pallas as pl
from jax.experimental.pallas import tpu as pltpu

T = 256
D = 512
V_HALF = 4096


def kernel(x, W):
    def body(x_ref, w_ref, out_ref, logits_buf, recv_buf, send_sem, recv_sem):
        my_x = lax.axis_index("x")
        my_y = lax.axis_index("y")
        my_z = lax.axis_index("z")
        partner = (my_x, 1 - my_y, my_z)

        barrier = pltpu.get_barrier_semaphore()
        pl.semaphore_signal(
            barrier, inc=1, device_id=partner,
            device_id_type=pl.DeviceIdType.MESH,
        )
        pl.semaphore_wait(barrier, 1)

        logits_buf[...] = jnp.dot(
            x_ref[...], w_ref[...], preferred_element_type=jnp.float32
        )

        rdma = pltpu.make_async_remote_copy(
            src_ref=logits_buf,
            dst_ref=recv_buf,
            send_sem=send_sem,
            recv_sem=recv_sem,
            device_id=partner,
            device_id_type=pl.DeviceIdType.MESH,
        )
        rdma.start()
        rdma.wait()

        out_ref[:, pl.ds(my_y * V_HALF, V_HALF)] = logits_buf[...]
        out_ref[:, pl.ds((1 - my_y) * V_HALF, V_HALF)] = recv_buf[...]
        l = out_ref[...]
        m = jnp.max(l, axis=1, keepdims=True)
        e = jnp.exp(l - m)
        s = jnp.sum(e, axis=1, keepdims=True)
        out_ref[...] = e / s

    return pl.pallas_call(
        body,
        out_shape=jax.ShapeDtypeStruct((T, 2 * V_HALF), jnp.float32),
        in_specs=[
            pl.BlockSpec(memory_space=pltpu.VMEM),
            pl.BlockSpec(memory_space=pltpu.VMEM),
        ],
        out_specs=pl.BlockSpec(memory_space=pltpu.VMEM),
        scratch_shapes=[
            pltpu.VMEM((T, V_HALF), jnp.float32),
            pltpu.VMEM((T, V_HALF), jnp.float32),
            pltpu.SemaphoreType.DMA,
            pltpu.SemaphoreType.DMA,
        ],
        compiler_params=pltpu.CompilerParams(collective_id=0),
    )(x, W)


# device time: 58067 ns/iter; 1.0545x vs baseline; 1.0545x over previous
import jax
import jax.numpy as jnp
from jax import lax
from jax.experimental import pallas as pl
from jax.experimental.pallas import tpu as pltpu

T = 256
D = 512
V_HALF = 4096
K = 4
C = V_HALF // K


def kernel(x, W):
    def body(x_ref, w_ref, out_ref, logits_buf, recv_buf, send_sems, recv_sems):
        my_x = lax.axis_index("x")
        my_y = lax.axis_index("y")
        my_z = lax.axis_index("z")
        partner = (my_x, 1 - my_y, my_z)

        barrier = pltpu.get_barrier_semaphore()
        pl.semaphore_signal(
            barrier, inc=1, device_id=partner,
            device_id_type=pl.DeviceIdType.MESH,
        )
        pl.semaphore_wait(barrier, 1)

        def chunk_rdma(k):
            return pltpu.make_async_remote_copy(
                src_ref=logits_buf.at[:, pl.ds(k * C, C)],
                dst_ref=recv_buf.at[:, pl.ds(k * C, C)],
                send_sem=send_sems.at[k],
                recv_sem=recv_sems.at[k],
                device_id=partner,
                device_id_type=pl.DeviceIdType.MESH,
            )

        for k in range(K):
            logits_buf[:, pl.ds(k * C, C)] = jnp.dot(
                x_ref[...], w_ref[:, pl.ds(k * C, C)],
                preferred_element_type=jnp.float32,
            )
            chunk_rdma(k).start()

        e_loc = jnp.exp(logits_buf[...])
        out_ref[:, pl.ds(my_y * V_HALF, V_HALF)] = e_loc
        s = jnp.sum(e_loc, axis=1, keepdims=True)

        rem = (1 - my_y) * V_HALF
        for k in range(K):
            chunk_rdma(k).wait_recv()
            e_k = jnp.exp(recv_buf[:, pl.ds(k * C, C)])
            out_ref[:, pl.ds(rem + k * C, C)] = e_k
            s = s + jnp.sum(e_k, axis=1, keepdims=True)

        out_ref[...] = out_ref[...] * (1.0 / s)

        for k in range(K):
            chunk_rdma(k).wait_send()

    return pl.pallas_call(
        body,
        out_shape=jax.ShapeDtypeStruct((T, 2 * V_HALF), jnp.float32),
        in_specs=[
            pl.BlockSpec(memory_space=pltpu.VMEM),
            pl.BlockSpec(memory_space=pltpu.VMEM),
        ],
        out_specs=pl.BlockSpec(memory_space=pltpu.VMEM),
        scratch_shapes=[
            pltpu.VMEM((T, V_HALF), jnp.float32),
            pltpu.VMEM((T, V_HALF), jnp.float32),
            pltpu.SemaphoreType.DMA((K,)),
            pltpu.SemaphoreType.DMA((K,)),
        ],
        compiler_params=pltpu.CompilerParams(collective_id=0),
    )(x, W)


# device time: 46410 ns/iter; 1.3193x vs baseline; 1.2512x over previous
import jax
import jax.numpy as jnp
from jax import lax
from jax.experimental import pallas as pl
from jax.experimental.pallas import tpu as pltpu

T = 256
D = 512
V_HALF = 4096
HALF_R = 128
NSUB = 4
SUB = HALF_R // NSUB
COL_H = V_HALF // 2


def kernel(x, W):
    def body(x_ref, w_ref, out_ref, e_loc, R, r_buf, s_send_buf, s_recv_buf,
             y_send_sems, y_recv_sems, xf_send_sems, x_recv_sems,
             zf_send_sems, z_recv_sems, s_send_sem, s_recv_sem):
        my_x = lax.axis_index("x")
        my_y = lax.axis_index("y")
        my_z = lax.axis_index("z")
        P = (my_x, 1 - my_y, my_z)
        X = (1 - my_x, my_y, my_z)
        Z = (my_x, my_y, 1 - my_z)
        p = lax.rem(my_x + my_z, 2)
        half0 = p * HALF_R
        half1 = (1 - p) * HALF_R

        barrier = pltpu.get_barrier_semaphore()
        for nbr in (P, X, Z):
            pl.semaphore_signal(
                barrier, inc=1, device_id=nbr,
                device_id_type=pl.DeviceIdType.MESH,
            )
        pl.semaphore_wait(barrier, 3)

        def rows(base, i):
            return pl.ds(base + i * SUB, SUB)

        def y_rdma(i):
            return pltpu.make_async_remote_copy(
                src_ref=e_loc.at[rows(half0, i), :],
                dst_ref=R.at[rows(half0, i), :],
                send_sem=y_send_sems.at[i],
                recv_sem=y_recv_sems.at[i],
                device_id=P,
                device_id_type=pl.DeviceIdType.MESH,
            )

        def xf_rdma(i):
            return pltpu.make_async_remote_copy(
                src_ref=R.at[rows(half0, i), pl.ds(0, COL_H)],
                dst_ref=R.at[rows(half0, i), pl.ds(0, COL_H)],
                send_sem=xf_send_sems.at[i],
                recv_sem=x_recv_sems.at[i],
                device_id=X,
                device_id_type=pl.DeviceIdType.MESH,
            )

        def zf_rdma(i):
            return pltpu.make_async_remote_copy(
                src_ref=R.at[rows(half0, i), pl.ds(COL_H, COL_H)],
                dst_ref=R.at[rows(half0, i), pl.ds(COL_H, COL_H)],
                send_sem=zf_send_sems.at[i],
                recv_sem=z_recv_sems.at[i],
                device_id=Z,
                device_id_type=pl.DeviceIdType.MESH,
            )

        def x_recv(i):
            return pltpu.make_async_remote_copy(
                src_ref=R.at[rows(half1, i), pl.ds(0, COL_H)],
                dst_ref=R.at[rows(half1, i), pl.ds(0, COL_H)],
                send_sem=xf_send_sems.at[i],
                recv_sem=x_recv_sems.at[i],
                device_id=X,
                device_id_type=pl.DeviceIdType.MESH,
            )

        def z_recv(i):
            return pltpu.make_async_remote_copy(
                src_ref=R.at[rows(half1, i), pl.ds(COL_H, COL_H)],
                dst_ref=R.at[rows(half1, i), pl.ds(COL_H, COL_H)],
                send_sem=zf_send_sems.at[i],
                recv_sem=z_recv_sems.at[i],
                device_id=Z,
                device_id_type=pl.DeviceIdType.MESH,
            )

        for piece in range(2):
            rws = pl.ds(half0 + piece * 2 * SUB, 2 * SUB)
            e_loc[rws, :] = jnp.exp(jnp.dot(
                x_ref[rws, :], w_ref[...],
                preferred_element_type=jnp.float32,
            ))
            y_rdma(2 * piece).start()
            y_rdma(2 * piece + 1).start()
        rws = pl.ds(half1, HALF_R)
        e_loc[rws, :] = jnp.exp(jnp.dot(
            x_ref[rws, :], w_ref[...], preferred_element_type=jnp.float32,
        ))

        s_loc = jnp.sum(e_loc[...], axis=1, keepdims=True)
        s_send_buf[...] = jnp.broadcast_to(s_loc, (T, 128))
        s_rdma = pltpu.make_async_remote_copy(
            src_ref=s_send_buf,
            dst_ref=s_recv_buf,
            send_sem=s_send_sem,
            recv_sem=s_recv_sem,
            device_id=P,
            device_id_type=pl.DeviceIdType.MESH,
        )
        s_rdma.start()

        for i in range(2):
            y_rdma(i).wait_recv()
            xf_rdma(i).start()
            zf_rdma(i).start()

        s_rdma.wait_recv()
        r = 1.0 / (s_loc + s_recv_buf[:, 0:1])
        r_buf[...] = jnp.broadcast_to(r, (T, 128))
        loc = my_y * V_HALF
        rem = (1 - my_y) * V_HALF
        out_ref[:, pl.ds(loc, V_HALF)] = e_loc[...] * r

        for i in range(2, NSUB):
            y_rdma(i).wait_recv()
            xf_rdma(i).start()
            zf_rdma(i).start()

        out_ref[pl.ds(half0, HALF_R), pl.ds(rem, V_HALF)] = (
            R[pl.ds(half0, HALF_R), :] * r_buf[pl.ds(half0, HALF_R), 0:1]
        )

        for i in range(NSUB):
            x_recv(i).wait_recv()
            out_ref[rows(half1, i), pl.ds(rem, COL_H)] = (
                R[rows(half1, i), pl.ds(0, COL_H)]
                * r_buf[rows(half1, i), 0:1]
            )
            z_recv(i).wait_recv()
            out_ref[rows(half1, i), pl.ds(rem + COL_H, COL_H)] = (
                R[rows(half1, i), pl.ds(COL_H, COL_H)]
                * r_buf[rows(half1, i), 0:1]
            )

        for i in range(NSUB):
            y_rdma(i).wait_send()
            xf_rdma(i).wait_send()
            zf_rdma(i).wait_send()
        s_rdma.wait_send()

    return pl.pallas_call(
        body,
        out_shape=jax.ShapeDtypeStruct((T, 2 * V_HALF), jnp.float32),
        in_specs=[
            pl.BlockSpec(memory_space=pltpu.VMEM),
            pl.BlockSpec(memory_space=pltpu.VMEM),
        ],
        out_specs=pl.BlockSpec(memory_space=pltpu.VMEM),
        scratch_shapes=[
            pltpu.VMEM((T, V_HALF), jnp.float32),
            pltpu.VMEM((T, V_HALF), jnp.float32),
            pltpu.VMEM((T, 128), jnp.float32),
            pltpu.VMEM((T, 128), jnp.float32),
            pltpu.VMEM((T, 128), jnp.float32),
            pltpu.SemaphoreType.DMA((NSUB,)),
            pltpu.SemaphoreType.DMA((NSUB,)),
            pltpu.SemaphoreType.DMA((NSUB,)),
            pltpu.SemaphoreType.DMA((NSUB,)),
            pltpu.SemaphoreType.DMA((NSUB,)),
            pltpu.SemaphoreType.DMA((NSUB,)),
            pltpu.SemaphoreType.DMA,
            pltpu.SemaphoreType.DMA,
        ],
        compiler_params=pltpu.CompilerParams(collective_id=0),
    )(x, W)


# device time: 32312 ns/iter; 1.8949x vs baseline; 1.4363x over previous
import jax
import jax.numpy as jnp
from jax import lax
from jax.experimental import pallas as pl
from jax.experimental.pallas import tpu as pltpu

T = 256
D = 512
V_HALF = 4096
HALF_R = 128
NSUB = 4
SUB = HALF_R // NSUB
COL_H = V_HALF // 2


def kernel(x, W):
    def body(x_ref, w_ref, out_ref, e_loc, e_bf, R, x_bf, w_bf, r_buf,
             s_send_buf, s_recv_buf,
             y_send_sems, y_recv_sems, xf_send_sems, x_recv_sems,
             zf_send_sems, z_recv_sems, s_send_sem, s_recv_sem):
        my_x = lax.axis_index("x")
        my_y = lax.axis_index("y")
        my_z = lax.axis_index("z")
        P = (my_x, 1 - my_y, my_z)
        X = (1 - my_x, my_y, my_z)
        Z = (my_x, my_y, 1 - my_z)
        p = lax.rem(my_x + my_z, 2)
        half0 = p * HALF_R
        half1 = (1 - p) * HALF_R

        barrier = pltpu.get_barrier_semaphore()
        for nbr in (P, X, Z):
            pl.semaphore_signal(
                barrier, inc=1, device_id=nbr,
                device_id_type=pl.DeviceIdType.MESH,
            )
        pl.semaphore_wait(barrier, 3)

        def rows(base, i):
            return pl.ds(base + i * SUB, SUB)

        def y_rdma(i):
            return pltpu.make_async_remote_copy(
                src_ref=e_bf.at[rows(half0, i), :],
                dst_ref=R.at[rows(half0, i), :],
                send_sem=y_send_sems.at[i],
                recv_sem=y_recv_sems.at[i],
                device_id=P,
                device_id_type=pl.DeviceIdType.MESH,
            )

        def xf_rdma(i):
            return pltpu.make_async_remote_copy(
                src_ref=R.at[rows(half0, i), pl.ds(0, COL_H)],
                dst_ref=R.at[rows(half0, i), pl.ds(0, COL_H)],
                send_sem=xf_send_sems.at[i],
                recv_sem=x_recv_sems.at[i],
                device_id=X,
                device_id_type=pl.DeviceIdType.MESH,
            )

        def zf_rdma(i):
            return pltpu.make_async_remote_copy(
                src_ref=R.at[rows(half0, i), pl.ds(COL_H, COL_H)],
                dst_ref=R.at[rows(half0, i), pl.ds(COL_H, COL_H)],
                send_sem=zf_send_sems.at[i],
                recv_sem=z_recv_sems.at[i],
                device_id=Z,
                device_id_type=pl.DeviceIdType.MESH,
            )

        def x_recv(i):
            return pltpu.make_async_remote_copy(
                src_ref=R.at[rows(half1, i), pl.ds(0, COL_H)],
                dst_ref=R.at[rows(half1, i), pl.ds(0, COL_H)],
                send_sem=xf_send_sems.at[i],
                recv_sem=x_recv_sems.at[i],
                device_id=X,
                device_id_type=pl.DeviceIdType.MESH,
            )

        def z_recv(i):
            return pltpu.make_async_remote_copy(
                src_ref=R.at[rows(half1, i), pl.ds(COL_H, COL_H)],
                dst_ref=R.at[rows(half1, i), pl.ds(COL_H, COL_H)],
                send_sem=zf_send_sems.at[i],
                recv_sem=z_recv_sems.at[i],
                device_id=Z,
                device_id_type=pl.DeviceIdType.MESH,
            )

        x_bf[...] = x_ref[...].astype(jnp.bfloat16)
        w_bf[...] = w_ref[...].astype(jnp.bfloat16)

        pieces = [(0, SUB, [0]), (SUB, SUB, [1]), (2 * SUB, 2 * SUB, [2, 3])]
        for off, n, subs in pieces:
            rws = pl.ds(half0 + off, n)
            e = jnp.exp(jnp.dot(
                x_bf[rws, :], w_bf[...],
                preferred_element_type=jnp.float32,
            ))
            e_loc[rws, :] = e
            e_bf[rws, :] = e.astype(jnp.bfloat16)
            for i in subs:
                y_rdma(i).start()
        rws = pl.ds(half1, HALF_R)
        e_loc[rws, :] = jnp.exp(jnp.dot(
            x_bf[rws, :], w_bf[...], preferred_element_type=jnp.float32,
        ))

        s_loc = jnp.sum(e_loc[...], axis=1, keepdims=True)
        s_send_buf[...] = jnp.broadcast_to(s_loc, (T, 128))
        s_rdma = pltpu.make_async_remote_copy(
            src_ref=s_send_buf,
            dst_ref=s_recv_buf,
            send_sem=s_send_sem,
            recv_sem=s_recv_sem,
            device_id=P,
            device_id_type=pl.DeviceIdType.MESH,
        )
        s_rdma.start()

        for i in range(2):
            y_rdma(i).wait_recv()
            xf_rdma(i).start()
            zf_rdma(i).start()

        s_rdma.wait_recv()
        r = 1.0 / (s_loc + s_recv_buf[:, 0:1])
        r_buf[...] = jnp.broadcast_to(r, (T, 128))
        loc = my_y * V_HALF
        rem = (1 - my_y) * V_HALF
        out_ref[:, pl.ds(loc, V_HALF)] = e_loc[...] * r

        for i in range(2, NSUB):
            y_rdma(i).wait_recv()
            xf_rdma(i).start()
            zf_rdma(i).start()

        out_ref[pl.ds(half0, HALF_R), pl.ds(rem, V_HALF)] = (
            R[pl.ds(half0, HALF_R), :].astype(jnp.float32)
            * r_buf[pl.ds(half0, HALF_R), 0:1]
        )

        for i in range(NSUB):
            x_recv(i).wait_recv()
            out_ref[rows(half1, i), pl.ds(rem, COL_H)] = (
                R[rows(half1, i), pl.ds(0, COL_H)].astype(jnp.float32)
                * r_buf[rows(half1, i), 0:1]
            )
            z_recv(i).wait_recv()
            out_ref[rows(half1, i), pl.ds(rem + COL_H, COL_H)] = (
                R[rows(half1, i), pl.ds(COL_H, COL_H)].astype(jnp.float32)
                * r_buf[rows(half1, i), 0:1]
            )

        for i in range(NSUB):
            y_rdma(i).wait_send()
            xf_rdma(i).wait_send()
            zf_rdma(i).wait_send()
        s_rdma.wait_send()

    return pl.pallas_call(
        body,
        out_shape=jax.ShapeDtypeStruct((T, 2 * V_HALF), jnp.float32),
        in_specs=[
            pl.BlockSpec(memory_space=pltpu.VMEM),
            pl.BlockSpec(memory_space=pltpu.VMEM),
        ],
        out_specs=pl.BlockSpec(memory_space=pltpu.VMEM),
        scratch_shapes=[
            pltpu.VMEM((T, V_HALF), jnp.float32),
            pltpu.VMEM((T, V_HALF), jnp.bfloat16),
            pltpu.VMEM((T, V_HALF), jnp.bfloat16),
            pltpu.VMEM((T, D), jnp.bfloat16),
            pltpu.VMEM((D, V_HALF), jnp.bfloat16),
            pltpu.VMEM((T, 128), jnp.float32),
            pltpu.VMEM((T, 128), jnp.float32),
            pltpu.VMEM((T, 128), jnp.float32),
            pltpu.SemaphoreType.DMA((NSUB,)),
            pltpu.SemaphoreType.DMA((NSUB,)),
            pltpu.SemaphoreType.DMA((NSUB,)),
            pltpu.SemaphoreType.DMA((NSUB,)),
            pltpu.SemaphoreType.DMA((NSUB,)),
            pltpu.SemaphoreType.DMA((NSUB,)),
            pltpu.SemaphoreType.DMA,
            pltpu.SemaphoreType.DMA,
        ],
        compiler_params=pltpu.CompilerParams(collective_id=0),
    )(x, W)


# device time: 27819 ns/iter; 2.2010x vs baseline; 1.1615x over previous
import jax
import jax.numpy as jnp
from jax import lax
from jax.experimental import pallas as pl
from jax.experimental.pallas import tpu as pltpu

T = 256
D = 512
V_HALF = 4096
QR = 64
NS = 2
SUBQ = QR // NS
COL_H = V_HALF // 2


def kernel(x, W):
    def body(x_ref, w_ref, out_ref, e_bf, R, x_bf, w_bf, r_buf,
             s_send_buf, s_recv_buf,
             y_s, y_r, xf_s, xf_r, zf_s, zf_r, xd_s, xd_r, zd_s, zd_r,
             s_send_sem, s_recv_sem):
        my_x = lax.axis_index("x")
        my_y = lax.axis_index("y")
        my_z = lax.axis_index("z")
        P = (my_x, 1 - my_y, my_z)
        X = (1 - my_x, my_y, my_z)
        Z = (my_x, my_y, 1 - my_z)
        q = 2 * my_x + my_z
        qx = 2 * (1 - my_x) + my_z
        qz = 2 * my_x + (1 - my_z)
        qd = 2 * (1 - my_x) + (1 - my_z)

        barrier = pltpu.get_barrier_semaphore()
        for nbr in (P, X, Z):
            pl.semaphore_signal(
                barrier, inc=1, device_id=nbr,
                device_id_type=pl.DeviceIdType.MESH,
            )
        pl.semaphore_wait(barrier, 3)

        def rows(quarter, i):
            return pl.ds(quarter * QR + i * SUBQ, SUBQ)

        def mk(src, dst, send_sem, recv_sem, dev):
            return pltpu.make_async_remote_copy(
                src_ref=src, dst_ref=dst, send_sem=send_sem,
                recv_sem=recv_sem, device_id=dev,
                device_id_type=pl.DeviceIdType.MESH,
            )

        def y_rdma(i):
            return mk(e_bf.at[rows(q, i), :], R.at[rows(q, i), :],
                      y_s.at[i], y_r.at[i], P)

        def xf_rdma(i):
            return mk(R.at[rows(q, i), :], R.at[rows(q, i), :],
                      xf_s.at[i], xf_r.at[i], X)

        def zf_rdma(i):
            return mk(R.at[rows(q, i), :], R.at[rows(q, i), :],
                      zf_s.at[i], zf_r.at[i], Z)

        def x_fwd_in(i):
            return mk(R.at[rows(qx, i), :], R.at[rows(qx, i), :],
                      xf_s.at[i], xf_r.at[i], X)

        def z_fwd_in(i):
            return mk(R.at[rows(qz, i), :], R.at[rows(qz, i), :],
                      zf_s.at[i], zf_r.at[i], Z)

        def xd_out(i):
            return mk(R.at[rows(qz, i), pl.ds(0, COL_H)],
                      R.at[rows(qz, i), pl.ds(0, COL_H)],
                      xd_s.at[i], xd_r.at[i], X)

        def zd_out(i):
            return mk(R.at[rows(qx, i), pl.ds(COL_H, COL_H)],
                      R.at[rows(qx, i), pl.ds(COL_H, COL_H)],
                      zd_s.at[i], zd_r.at[i], Z)

        def xd_in(i):
            return mk(R.at[rows(qd, i), pl.ds(0, COL_H)],
                      R.at[rows(qd, i), pl.ds(0, COL_H)],
                      xd_s.at[i], xd_r.at[i], X)

        def zd_in(i):
            return mk(R.at[rows(qd, i), pl.ds(COL_H, COL_H)],
                      R.at[rows(qd, i), pl.ds(COL_H, COL_H)],
                      zd_s.at[i], zd_r.at[i], Z)

        x_bf[...] = x_ref[...].astype(jnp.bfloat16)
        w_bf[...] = w_ref[...].astype(jnp.bfloat16)

        for j in range(4):
            qq = lax.rem(q + j, 4)
            rws = pl.ds(qq * QR, QR)
            e = jnp.exp(jnp.dot(
                x_bf[rws, :], w_bf[...],
                preferred_element_type=jnp.float32,
            ))
            e_bf[rws, :] = e.astype(jnp.bfloat16)
            s_send_buf[rws, :] = jnp.broadcast_to(
                jnp.sum(e, axis=1, keepdims=True), (QR, 128)
            )
            if j == 0:
                y_rdma(0).start()
                y_rdma(1).start()

        s_rdma = mk(s_send_buf, s_recv_buf, s_send_sem, s_recv_sem, P)
        s_rdma.start()

        for i in range(NS):
            y_rdma(i).wait_recv()
            xf_rdma(i).start()
            zf_rdma(i).start()

        s_rdma.wait_recv()
        r = 1.0 / (s_send_buf[:, 0:1] + s_recv_buf[:, 0:1])
        r_buf[...] = jnp.broadcast_to(r, (T, 128))
        loc = my_y * V_HALF
        rem = (1 - my_y) * V_HALF
        out_ref[:, pl.ds(loc, V_HALF)] = e_bf[...].astype(jnp.float32) * r
        out_ref[pl.ds(q * QR, QR), pl.ds(rem, V_HALF)] = (
            R[pl.ds(q * QR, QR), :].astype(jnp.float32)
            * r_buf[pl.ds(q * QR, QR), 0:1]
        )

        for i in range(NS):
            x_fwd_in(i).wait_recv()
            zd_out(i).start()
            out_ref[rows(qx, i), pl.ds(rem, V_HALF)] = (
                R[rows(qx, i), :].astype(jnp.float32)
                * r_buf[rows(qx, i), 0:1]
            )
            z_fwd_in(i).wait_recv()
            xd_out(i).start()
            out_ref[rows(qz, i), pl.ds(rem, V_HALF)] = (
                R[rows(qz, i), :].astype(jnp.float32)
                * r_buf[rows(qz, i), 0:1]
            )

        for i in range(NS):
            xd_in(i).wait_recv()
            out_ref[rows(qd, i), pl.ds(rem, COL_H)] = (
                R[rows(qd, i), pl.ds(0, COL_H)].astype(jnp.float32)
                * r_buf[rows(qd, i), 0:1]
            )
            zd_in(i).wait_recv()
            out_ref[rows(qd, i), pl.ds(rem + COL_H, COL_H)] = (
                R[rows(qd, i), pl.ds(COL_H, COL_H)].astype(jnp.float32)
                * r_buf[rows(qd, i), 0:1]
            )

        for i in range(NS):
            y_rdma(i).wait_send()
            xf_rdma(i).wait_send()
            zf_rdma(i).wait_send()
            xd_out(i).wait_send()
            zd_out(i).wait_send()
        s_rdma.wait_send()

    return pl.pallas_call(
        body,
        out_shape=jax.ShapeDtypeStruct((T, 2 * V_HALF), jnp.float32),
        in_specs=[
            pl.BlockSpec(memory_space=pltpu.VMEM),
            pl.BlockSpec(memory_space=pltpu.VMEM),
        ],
        out_specs=pl.BlockSpec(memory_space=pltpu.VMEM),
        scratch_shapes=[
            pltpu.VMEM((T, V_HALF), jnp.bfloat16),
            pltpu.VMEM((T, V_HALF), jnp.bfloat16),
            pltpu.VMEM((T, D), jnp.bfloat16),
            pltpu.VMEM((D, V_HALF), jnp.bfloat16),
            pltpu.VMEM((T, 128), jnp.float32),
            pltpu.VMEM((T, 128), jnp.float32),
            pltpu.VMEM((T, 128), jnp.float32),
            pltpu.SemaphoreType.DMA((NS,)),
            pltpu.SemaphoreType.DMA((NS,)),
            pltpu.SemaphoreType.DMA((NS,)),
            pltpu.SemaphoreType.DMA((NS,)),
            pltpu.SemaphoreType.DMA((NS,)),
            pltpu.SemaphoreType.DMA((NS,)),
            pltpu.SemaphoreType.DMA((NS,)),
            pltpu.SemaphoreType.DMA((NS,)),
            pltpu.SemaphoreType.DMA((NS,)),
            pltpu.SemaphoreType.DMA((NS,)),
            pltpu.SemaphoreType.DMA,
            pltpu.SemaphoreType.DMA,
        ],
        compiler_params=pltpu.CompilerParams(collective_id=0),
    )(x, W)
